# baseline (speedup 1.0000x reference)
import functools

import jax
import jax.numpy as jnp
from jax import lax
from jax.experimental import pallas as pl
from jax.experimental.pallas import tpu as pltpu
from jax.experimental.pallas import tpu_sc as plsc

_BATCH = 16
_N_FREQ = 8192
_N_LOW = 2048
_BLK = _N_FREQ // 128
_BLK_LOW = _N_LOW // 128
_WORDS = _BATCH * _N_FREQ * 2
_WORDS_LOW = _BATCH * _N_LOW * 2
_PER_BATCH = _N_FREQ * 2
_PER_BATCH_LOW = _N_LOW * 2
_HALF = _PER_BATCH // 2

_mesh = plsc.VectorSubcoreMesh(core_axis_name="c", subcore_axis_name="s")


_CHUNK = _PER_BATCH_LOW // 2


@functools.partial(
    pl.kernel,
    out_type=jax.ShapeDtypeStruct((_WORDS_LOW,), jnp.float32),
    mesh=_mesh,
    scratch_types=[pltpu.VMEM((_CHUNK,), jnp.float32)],
)
def _down_sample_sc(in_hbm, low_hbm, buf):
    wid = lax.axis_index("s") * 2 + lax.axis_index("c")
    batch = wid // 2
    half = wid % 2
    src = batch * _PER_BATCH + half * _CHUNK
    dst = batch * _PER_BATCH_LOW + half * _CHUNK
    pltpu.sync_copy(in_hbm.at[pl.ds(src, _CHUNK)], buf)
    pltpu.sync_copy(buf, low_hbm.at[pl.ds(dst, _CHUNK)])


def _to_flat(x):
    return (
        x.reshape(_BATCH, _BLK, 128, 2)
        .transpose(0, 1, 3, 2)
        .reshape(_WORDS)
    )


def _low_from_flat(flat):
    return (
        flat.reshape(_BATCH, _BLK_LOW, 2, 128)
        .transpose(0, 1, 3, 2)
        .reshape(_BATCH, _N_LOW, 2)
    )


def kernel(full_freq_info):
    low_flat = _down_sample_sc(_to_flat(full_freq_info))
    return (full_freq_info, _low_from_flat(low_flat))


# speedup vs baseline: 1.0219x; 1.0219x over previous
import functools

import jax
import jax.numpy as jnp
from jax import lax
from jax.experimental import pallas as pl
from jax.experimental.pallas import tpu as pltpu
from jax.experimental.pallas import tpu_sc as plsc

_BATCH = 16
_N_FREQ = 8192
_N_LOW = 2048
_BLK = _N_FREQ // 128
_BLK_LOW = _N_LOW // 128
_WORDS = _BATCH * _N_FREQ * 2
_WORDS_LOW = _BATCH * _N_LOW * 2
_PER_BATCH = _N_FREQ * 2
_PER_BATCH_LOW = _N_LOW * 2
_HALF = _PER_BATCH // 2

_mesh = plsc.VectorSubcoreMesh(core_axis_name="c", subcore_axis_name="s")


@functools.partial(
    pl.kernel,
    out_type=(
        jax.ShapeDtypeStruct((_WORDS,), jnp.float32),
        jax.ShapeDtypeStruct((_WORDS_LOW,), jnp.float32),
    ),
    mesh=_mesh,
    scratch_types=[
        pltpu.VMEM((_HALF,), jnp.float32),
        pltpu.SemaphoreType.DMA,
    ],
)
def _down_sample_sc(in_hbm, full_hbm, low_hbm, buf, sem):
    wid = lax.axis_index("s") * 2 + lax.axis_index("c")
    batch = wid // 2
    half = wid % 2
    src = batch * _PER_BATCH + half * _HALF
    pltpu.sync_copy(in_hbm.at[pl.ds(src, _HALF)], buf)
    back = pltpu.async_copy(buf, full_hbm.at[pl.ds(src, _HALF)], sem)

    @pl.when(half == 0)
    def _():
        pltpu.async_copy(
            buf.at[pl.ds(0, _PER_BATCH_LOW)],
            low_hbm.at[pl.ds(batch * _PER_BATCH_LOW, _PER_BATCH_LOW)],
            sem,
        ).wait()

    back.wait()


def _to_flat(x):
    return (
        x.reshape(_BATCH, _BLK, 128, 2)
        .transpose(0, 1, 3, 2)
        .reshape(_WORDS)
    )


def _full_from_flat(flat):
    return (
        flat.reshape(_BATCH, _BLK, 2, 128)
        .transpose(0, 1, 3, 2)
        .reshape(_BATCH, _N_FREQ, 2)
    )


def _low_from_flat(flat):
    return (
        flat.reshape(_BATCH, _BLK_LOW, 2, 128)
        .transpose(0, 1, 3, 2)
        .reshape(_BATCH, _N_LOW, 2)
    )


def kernel(full_freq_info):
    full_flat, low_flat = _down_sample_sc(_to_flat(full_freq_info))
    return (_full_from_flat(full_flat), _low_from_flat(low_flat))


# speedup vs baseline: 1.0567x; 1.0340x over previous
import functools

import jax
import jax.numpy as jnp
from jax import lax
from jax.experimental import pallas as pl
from jax.experimental.pallas import tpu as pltpu
from jax.experimental.pallas import tpu_sc as plsc

_BATCH = 16
_N_FREQ = 8192
_N_LOW = 2048
_BLK = _N_FREQ // 128
_BLK_LOW = _N_LOW // 128
_WORDS = _BATCH * _N_FREQ * 2
_WORDS_LOW = _BATCH * _N_LOW * 2
_PER_BATCH = _N_FREQ * 2
_PER_BATCH_LOW = _N_LOW * 2
_HALF = _PER_BATCH // 2

_mesh = plsc.VectorSubcoreMesh(
    core_axis_name="c", subcore_axis_name="s", num_cores=1
)


@functools.partial(
    pl.kernel,
    out_type=(
        jax.ShapeDtypeStruct((_WORDS,), jnp.float32),
        jax.ShapeDtypeStruct((_WORDS_LOW,), jnp.float32),
    ),
    mesh=_mesh,
    scratch_types=[
        pltpu.VMEM((_PER_BATCH,), jnp.float32),
        pltpu.SemaphoreType.DMA,
    ],
)
def _down_sample_sc(in_hbm, full_hbm, low_hbm, buf, sem):
    batch = lax.axis_index("s")
    src = batch * _PER_BATCH
    pltpu.sync_copy(in_hbm.at[pl.ds(src, _PER_BATCH)], buf)
    back = pltpu.async_copy(buf, full_hbm.at[pl.ds(src, _PER_BATCH)], sem)
    pltpu.async_copy(
        buf.at[pl.ds(0, _PER_BATCH_LOW)],
        low_hbm.at[pl.ds(batch * _PER_BATCH_LOW, _PER_BATCH_LOW)],
        sem,
    ).wait()
    back.wait()


def _to_flat(x):
    return (
        x.reshape(_BATCH, _BLK, 128, 2)
        .transpose(0, 1, 3, 2)
        .reshape(_WORDS)
    )


def _full_from_flat(flat):
    return (
        flat.reshape(_BATCH, _BLK, 2, 128)
        .transpose(0, 1, 3, 2)
        .reshape(_BATCH, _N_FREQ, 2)
    )


def _low_from_flat(flat):
    return (
        flat.reshape(_BATCH, _BLK_LOW, 2, 128)
        .transpose(0, 1, 3, 2)
        .reshape(_BATCH, _N_LOW, 2)
    )


def kernel(full_freq_info):
    full_flat, low_flat = _down_sample_sc(_to_flat(full_freq_info))
    return (_full_from_flat(full_flat), _low_from_flat(low_flat))
